# baseline (speedup 1.0000x reference)
import jax
import jax.numpy as jnp
from jax.experimental import pallas as pl
from jax.experimental.pallas import tpu as pltpu

_CHUNK = 512
_NBUF = 8


def _router_body(x_hbm, wt_ref, b_ref, out_ref, bufs, sems):
    n_tokens = x_hbm.shape[0]
    n_chunks = n_tokens // _CHUNK

    def copy_in(g, slot):
        src = x_hbm.at[pl.ds(pl.multiple_of(g * _CHUNK, _CHUNK), _CHUNK)]
        return pltpu.make_async_copy(src, bufs.at[slot], sems.at[slot])

    for slot in range(_NBUF):
        copy_in(slot, slot).start()

    wt = wt_ref[...]
    bias = b_ref[...]

    @pl.loop(0, n_chunks, step=_NBUF)
    def outer(g0):
        for slot in range(_NBUF):
            g = g0 + slot
            copy_in(g, slot).wait()
            logits = jnp.dot(bufs[slot], wt,
                             preferred_element_type=jnp.float32)
            logits = logits + bias
            m = jnp.max(logits, axis=-1, keepdims=True)
            e = jnp.exp(logits - m)
            probs = e / jnp.sum(e, axis=-1, keepdims=True)
            out_ref[pl.ds(pl.multiple_of(g * _CHUNK, _CHUNK), _CHUNK), :] = probs

            @pl.when(g + _NBUF < n_chunks)
            def _():
                copy_in(g + _NBUF, slot).start()


def kernel(x, W, b):
    n_tokens, d = x.shape
    n_experts = W.shape[0]
    return pl.pallas_call(
        _router_body,
        in_specs=[
            pl.BlockSpec(memory_space=pl.ANY),
            pl.BlockSpec(memory_space=pltpu.VMEM),
            pl.BlockSpec(memory_space=pltpu.VMEM),
        ],
        out_specs=pl.BlockSpec(memory_space=pltpu.VMEM),
        out_shape=jax.ShapeDtypeStruct((n_tokens, n_experts), jnp.float32),
        scratch_shapes=[
            pltpu.VMEM((_NBUF, _CHUNK, d), jnp.float32),
            pltpu.SemaphoreType.DMA((_NBUF,)),
        ],
    )(x, W.T, b.reshape(1, n_experts))


# speedup vs baseline: 1.2964x; 1.2964x over previous
import jax
import jax.numpy as jnp
from jax.experimental import pallas as pl
from jax.experimental.pallas import tpu as pltpu

_BLOCK = 2048


def _router_block(x_ref, wt_ref, b_ref, out_ref):
    xb = x_ref[...].astype(jnp.bfloat16)
    logits = jnp.dot(xb, wt_ref[...], preferred_element_type=jnp.float32)
    logits = logits + b_ref[...]
    m = jnp.max(logits, axis=-1, keepdims=True)
    e = jnp.exp(logits - m)
    out_ref[...] = e / jnp.sum(e, axis=-1, keepdims=True)


def kernel(x, W, b):
    n_tokens, d = x.shape
    n_experts = W.shape[0]
    grid = (n_tokens // _BLOCK,)
    return pl.pallas_call(
        _router_block,
        grid=grid,
        in_specs=[
            pl.BlockSpec((_BLOCK, d), lambda i: (i, 0)),
            pl.BlockSpec((d, n_experts), lambda i: (0, 0)),
            pl.BlockSpec((1, n_experts), lambda i: (0, 0)),
        ],
        out_specs=pl.BlockSpec((_BLOCK, n_experts), lambda i: (i, 0)),
        out_shape=jax.ShapeDtypeStruct((n_tokens, n_experts), jnp.float32),
        compiler_params=pltpu.CompilerParams(
            dimension_semantics=("arbitrary",),
        ),
    )(x, W.T.astype(jnp.bfloat16), b.reshape(1, n_experts))
